# baseline (speedup 1.0000x reference)
import functools

import jax
import jax.numpy as jnp
from jax import lax
from jax.experimental import pallas as pl
from jax.experimental.pallas import tpu as pltpu
from jax.experimental.pallas import tpu_sc as plsc

EMBEDDING_SIZE = 64
BATCH = 16384
HIST_LEN = 50

_info = plsc.get_sparse_core_info()
_NC, _NS = _info.num_cores, _info.num_subcores
_NW = _NC * _NS
_NBB = BATCH // 128
_BB_W = _NBB // _NW
_TPAD = 136


def _make_gather():
    mesh = plsc.VectorSubcoreMesh(core_axis_name="c", subcore_axis_name="s")

    @functools.partial(
        pl.kernel,
        mesh=mesh,
        compiler_params=pltpu.CompilerParams(
            use_tc_tiling_on_sc=False, needs_layout_passes=False
        ),
        out_type=jax.ShapeDtypeStruct((HIST_LEN, 8, 128, 8, 128), jnp.float32),
        scratch_types=[
            pltpu.VMEM((6400,), jnp.int32),
            pltpu.VMEM((HIST_LEN, 128), jnp.int32),
            pltpu.VMEM((128, 128), jnp.float32),
            pltpu.VMEM((128, 128), jnp.float32),
            pltpu.VMEM((64, _TPAD), jnp.float32),
            pltpu.VMEM((64, _TPAD), jnp.float32),
            pltpu.SemaphoreType.DMA,
            pltpu.SemaphoreType.DMA,
        ],
    )
    def gather_kernel(idx_hbm, table_hbm, out_hbm, idx_v, idxt_v,
                      g0, g1, t0, t1, gsem, ssem):
        wid = lax.axis_index("s") * _NC + lax.axis_index("c")
        lane = lax.broadcasted_iota(jnp.int32, (16,), 0)

        def fire_gather(h, gb):
            return pltpu.async_copy(table_hbm.at[idxt_v.at[h]], gb, gsem)

        def drain_gather(gb):
            pltpu.make_async_copy(
                table_hbm.at[idxt_v.at[0]], gb, gsem
            ).wait()

        def fire_store(h, bb, tb):
            for do in range(8):
                pltpu.async_copy(
                    tb.at[pl.ds(do * 8, 8), pl.ds(0, 128)],
                    out_hbm.at[h, do, bb, :, :],
                    ssem,
                )

        def drain_store(tb):
            for do in range(8):
                pltpu.make_async_copy(
                    tb.at[pl.ds(do * 8, 8), pl.ds(0, 128)],
                    out_hbm.at[0, 0, 0, :, :],
                    ssem,
                ).wait()

        def transpose_tile(gb, tb):
            rows = [lane + (c * 16) for c in range(4)]

            @plsc.parallel_loop(0, 128)
            def tr_body(b):
                bcol = jnp.broadcast_to(b, (16,)).astype(jnp.int32)
                vs = [gb[b, pl.ds(c * 16, 16)] for c in range(4)]
                for c in range(4):
                    plsc.store_scatter(tb, [rows[c], bcol], vs[c])

        def do_block(bb, _):
            pltpu.sync_copy(idx_hbm.at[pl.ds(bb * 6400, 6400)], idx_v)

            @plsc.parallel_loop(0, HIST_LEN)
            def idx_t_body(h):
                hv = jnp.broadcast_to(h, (16,)).astype(jnp.int32)
                for c in range(8):
                    addr = (lane + (c * 16)) * HIST_LEN + hv
                    idxt_v[h, pl.ds(c * 16, 16)] = plsc.load_gather(
                        idx_v, [addr]
                    )

            fire_gather(0, g0)

            def do_pair(t, carry):
                h0 = 2 * t
                drain_gather(g0)
                fire_gather(h0 + 1, g1)

                @pl.when(t > 0)
                def _():
                    drain_store(t0)

                transpose_tile(g0, t0)
                fire_store(h0, bb, t0)
                drain_gather(g1)

                @pl.when(t < HIST_LEN // 2 - 1)
                def _():
                    fire_gather(h0 + 2, g0)

                @pl.when(t > 0)
                def _():
                    drain_store(t1)

                transpose_tile(g1, t1)
                fire_store(h0 + 1, bb, t1)
                return carry

            lax.fori_loop(0, HIST_LEN // 2, do_pair, 0)
            drain_store(t0)
            drain_store(t1)
            return _

        lax.fori_loop(wid * _BB_W, (wid + 1) * _BB_W, do_block, 0)

    return gather_kernel


_gather = _make_gather()


def kernel(indices_words, table):
    idx_flat = indices_words.astype(jnp.int32).reshape(-1)
    table_pad = jnp.pad(table, ((0, 7), (0, 64)))
    phys = _gather(idx_flat, table_pad)
    return phys.transpose(2, 4, 0, 1, 3).reshape(BATCH, HIST_LEN, EMBEDDING_SIZE)


# speedup vs baseline: 1.0216x; 1.0216x over previous
import functools

import jax
import jax.numpy as jnp
from jax import lax
from jax.experimental import pallas as pl
from jax.experimental.pallas import tpu as pltpu
from jax.experimental.pallas import tpu_sc as plsc

EMBEDDING_SIZE = 64
BATCH = 16384
HIST_LEN = 50

_info = plsc.get_sparse_core_info()
_NC, _NS = _info.num_cores, _info.num_subcores
_NW = _NC * _NS
_NBB = BATCH // 128
_BB_W = _NBB // _NW
_TPAD = 136


def _make_gather():
    mesh = plsc.VectorSubcoreMesh(core_axis_name="c", subcore_axis_name="s")

    @functools.partial(
        pl.kernel,
        mesh=mesh,
        compiler_params=pltpu.CompilerParams(
            use_tc_tiling_on_sc=False, needs_layout_passes=False
        ),
        out_type=jax.ShapeDtypeStruct((HIST_LEN, 8, 128, 8, 128), jnp.float32),
        scratch_types=[
            pltpu.VMEM((6400,), jnp.int32),
            pltpu.VMEM((HIST_LEN, 128), jnp.int32),
            pltpu.VMEM((128, 64), jnp.float32),
            pltpu.VMEM((128, 64), jnp.float32),
            pltpu.VMEM((64, _TPAD), jnp.float32),
            pltpu.VMEM((64, _TPAD), jnp.float32),
            pltpu.SemaphoreType.DMA,
            pltpu.SemaphoreType.DMA,
        ],
    )
    def gather_kernel(idx_hbm, table_hbm, out_hbm, idx_v, idxt_v,
                      g0, g1, t0, t1, gsem, ssem):
        wid = lax.axis_index("s") * _NC + lax.axis_index("c")
        lane = lax.broadcasted_iota(jnp.int32, (16,), 0)

        def fire_gather(h, gb):
            return pltpu.async_copy(table_hbm.at[idxt_v.at[h]], gb, gsem)

        def drain_gather(gb):
            pltpu.make_async_copy(
                table_hbm.at[idxt_v.at[0]], gb, gsem
            ).wait()

        def fire_store(h, bb, tb):
            for do in range(8):
                pltpu.async_copy(
                    tb.at[pl.ds(do * 8, 8), pl.ds(0, 128)],
                    out_hbm.at[h, do, bb, :, :],
                    ssem,
                )

        def drain_store(tb):
            for do in range(8):
                pltpu.make_async_copy(
                    tb.at[pl.ds(do * 8, 8), pl.ds(0, 128)],
                    out_hbm.at[0, 0, 0, :, :],
                    ssem,
                ).wait()

        def transpose_tile(gb, tb):
            rows = [lane + (c * 16) for c in range(4)]

            @plsc.parallel_loop(0, 128)
            def tr_body(b):
                bcol = jnp.broadcast_to(b, (16,)).astype(jnp.int32)
                vs = [gb[b, pl.ds(c * 16, 16)] for c in range(4)]
                for c in range(4):
                    plsc.store_scatter(tb, [rows[c], bcol], vs[c])

        def do_block(bb, _):
            pltpu.sync_copy(idx_hbm.at[pl.ds(bb * 6400, 6400)], idx_v)

            @plsc.parallel_loop(0, HIST_LEN)
            def idx_t_body(h):
                hv = jnp.broadcast_to(h, (16,)).astype(jnp.int32)
                for c in range(8):
                    addr = (lane + (c * 16)) * HIST_LEN + hv
                    idxt_v[h, pl.ds(c * 16, 16)] = plsc.load_gather(
                        idx_v, [addr]
                    )

            fire_gather(0, g0)

            def do_pair(t, carry):
                h0 = 2 * t
                drain_gather(g0)
                fire_gather(h0 + 1, g1)

                @pl.when(t > 0)
                def _():
                    drain_store(t0)

                transpose_tile(g0, t0)
                fire_store(h0, bb, t0)
                drain_gather(g1)

                @pl.when(t < HIST_LEN // 2 - 1)
                def _():
                    fire_gather(h0 + 2, g0)

                @pl.when(t > 0)
                def _():
                    drain_store(t1)

                transpose_tile(g1, t1)
                fire_store(h0 + 1, bb, t1)
                return carry

            lax.fori_loop(0, HIST_LEN // 2, do_pair, 0)
            drain_store(t0)
            drain_store(t1)
            return _

        lax.fori_loop(wid * _BB_W, (wid + 1) * _BB_W, do_block, 0)

    return gather_kernel


_gather = _make_gather()


def kernel(indices_words, table):
    idx_flat = indices_words.astype(jnp.int32).reshape(-1)
    phys = _gather(idx_flat, table)
    return phys.transpose(2, 4, 0, 1, 3).reshape(BATCH, HIST_LEN, EMBEDDING_SIZE)


# speedup vs baseline: 1.0270x; 1.0053x over previous
import functools

import jax
import jax.numpy as jnp
from jax import lax
from jax.experimental import pallas as pl
from jax.experimental.pallas import tpu as pltpu
from jax.experimental.pallas import tpu_sc as plsc

EMBEDDING_SIZE = 64
BATCH = 16384
HIST_LEN = 50

_info = plsc.get_sparse_core_info()
_NC, _NS = _info.num_cores, _info.num_subcores
_NW = _NC * _NS
_NBB = BATCH // 128
_BB_W = _NBB // _NW
_TPAD = 136


def _make_gather():
    mesh = plsc.VectorSubcoreMesh(core_axis_name="c", subcore_axis_name="s")

    @functools.partial(
        pl.kernel,
        mesh=mesh,
        compiler_params=pltpu.CompilerParams(
            use_tc_tiling_on_sc=False, needs_layout_passes=False
        ),
        out_type=jax.ShapeDtypeStruct((HIST_LEN, 8, 128, 8, 128), jnp.float32),
        scratch_types=[
            pltpu.VMEM((6400,), jnp.int32),
            pltpu.VMEM((HIST_LEN, 128), jnp.int32),
            pltpu.VMEM((128, 64), jnp.float32),
            pltpu.VMEM((128, 64), jnp.float32),
            pltpu.VMEM((64, _TPAD), jnp.float32),
            pltpu.VMEM((64, _TPAD), jnp.float32),
            pltpu.SemaphoreType.DMA,
            pltpu.SemaphoreType.DMA,
        ],
    )
    def gather_kernel(idx_hbm, table_hbm, out_hbm, idx_v, idxt_v,
                      g0, g1, t0, t1, gsem, ssem):
        wid = lax.axis_index("s") * _NC + lax.axis_index("c")
        lane = lax.broadcasted_iota(jnp.int32, (16,), 0)

        def fire_gather(h, gb):
            return pltpu.async_copy(table_hbm.at[idxt_v.at[h]], gb, gsem)

        def drain_gather(gb):
            pltpu.make_async_copy(
                table_hbm.at[idxt_v.at[0]], gb, gsem
            ).wait()

        def fire_store(h, bb, tb):
            for do in range(8):
                pltpu.async_copy(
                    tb.at[pl.ds(do * 8, 8), pl.ds(0, 128)],
                    out_hbm.at[h, do, bb, :, :],
                    ssem,
                )

        def drain_store(tb):
            for do in range(8):
                pltpu.make_async_copy(
                    tb.at[pl.ds(do * 8, 8), pl.ds(0, 128)],
                    out_hbm.at[0, 0, 0, :, :],
                    ssem,
                ).wait()

        def transpose_tile(gb, tb):
            rows = [lane + (c * 16) for c in range(4)]

            @plsc.parallel_loop(0, 128, unroll=4)
            def tr_body(b):
                bcol = jnp.broadcast_to(b, (16,)).astype(jnp.int32)
                vs = [gb[b, pl.ds(c * 16, 16)] for c in range(4)]
                for c in range(4):
                    plsc.store_scatter(tb, [rows[c], bcol], vs[c])

        def do_block(bb, _):
            pltpu.sync_copy(idx_hbm.at[pl.ds(bb * 6400, 6400)], idx_v)

            @plsc.parallel_loop(0, HIST_LEN, unroll=2)
            def idx_t_body(h):
                hv = jnp.broadcast_to(h, (16,)).astype(jnp.int32)
                for c in range(8):
                    addr = (lane + (c * 16)) * HIST_LEN + hv
                    idxt_v[h, pl.ds(c * 16, 16)] = plsc.load_gather(
                        idx_v, [addr]
                    )

            fire_gather(0, g0)

            def do_pair(t, carry):
                h0 = 2 * t
                drain_gather(g0)
                fire_gather(h0 + 1, g1)

                @pl.when(t > 0)
                def _():
                    drain_store(t0)

                transpose_tile(g0, t0)
                fire_store(h0, bb, t0)
                drain_gather(g1)

                @pl.when(t < HIST_LEN // 2 - 1)
                def _():
                    fire_gather(h0 + 2, g0)

                @pl.when(t > 0)
                def _():
                    drain_store(t1)

                transpose_tile(g1, t1)
                fire_store(h0 + 1, bb, t1)
                return carry

            lax.fori_loop(0, HIST_LEN // 2, do_pair, 0)
            drain_store(t0)
            drain_store(t1)
            return _

        lax.fori_loop(wid * _BB_W, (wid + 1) * _BB_W, do_block, 0)

    return gather_kernel


_gather = _make_gather()


def kernel(indices_words, table):
    idx_flat = indices_words.astype(jnp.int32).reshape(-1)
    phys = _gather(idx_flat, table)
    return phys.transpose(2, 4, 0, 1, 3).reshape(BATCH, HIST_LEN, EMBEDDING_SIZE)
